# baseline (speedup 1.0000x reference)
import functools

import jax
import jax.numpy as jnp
from jax import lax
from jax.experimental import pallas as pl
from jax.experimental.pallas import tpu as pltpu
from jax.experimental.pallas import tpu_sc as plsc

B = 16384
D = 64
V = 1000000
VQ = V // 4
NC = 2
NS = 16
NW = NC * NS
BPW = B // NW
NCHUNK = 8
CHUNK = BPW // NCHUNK
L = 16
GPC = CHUNK // L

NB = V // 128
TAIL = V - NB * 128


def _relayout_body(uT_ref, iT_ref, utail_ref, itail_ref, uq_ref, iq_ref,
                   inbuf_u, inbuf_i, tbuf_u, tbuf_i, sem_in, sem_out):
    c = lax.axis_index("c")
    s = lax.axis_index("s")
    wid = s * NC + c
    trip = 244 + jnp.where(wid < NB - 244 * NW, 1, 0)

    iota = lax.iota(jnp.int32, L)
    qv, sv, cb = [], [], []
    for g in range(8):
        u = g * L + iota
        qv.append(u >> 2)
        sv.append((u >> 1) & 1)
        cb.append((u & 1) * D)

    def fire_in(j, b):
        jc = jnp.minimum(j, NB - 1)
        pltpu.async_copy(uT_ref.at[:, pl.ds(jc * 128, 128)],
                         inbuf_u.at[b], sem_in)
        pltpu.async_copy(iT_ref.at[:, pl.ds(jc * 128, 128)],
                         inbuf_i.at[b], sem_in)

    def drain_in():
        pltpu.make_async_copy(uT_ref.at[:, pl.ds(0, 128)],
                              inbuf_u.at[0], sem_in).wait()
        pltpu.make_async_copy(iT_ref.at[:, pl.ds(0, 128)],
                              inbuf_i.at[0], sem_in).wait()

    def drain_out():
        pltpu.make_async_copy(uq_ref.at[pl.ds(0, 32)],
                              tbuf_u.at[0], sem_out).wait()
        pltpu.make_async_copy(iq_ref.at[pl.ds(0, 32)],
                              tbuf_i.at[0], sem_out).wait()

    def transpose_block(src, dst, ngroups):
        for d in range(D):
            dvec = jnp.full((L,), d, jnp.int32)
            for g in range(ngroups):
                vals = plsc.load_gather(src, [dvec, g * L + iota])
                plsc.store_scatter(dst, [qv[g], sv[g], cb[g] + d], vals)

    fire_in(wid, 0)

    def body(k, carry):
        b = k % 2
        fire_in(wid + NW * (k + 1), 1 - b)
        drain_in()

        @pl.when(k >= 2)
        def _():
            drain_out()

        j = wid + NW * k
        transpose_block(inbuf_u.at[b], tbuf_u.at[b], 8)
        transpose_block(inbuf_i.at[b], tbuf_i.at[b], 8)
        pltpu.async_copy(tbuf_u.at[b], uq_ref.at[pl.ds(j * 32, 32)], sem_out)
        pltpu.async_copy(tbuf_i.at[b], iq_ref.at[pl.ds(j * 32, 32)], sem_out)
        return carry

    lax.fori_loop(0, trip, body, 0)
    drain_in()

    @pl.when(trip >= 2)
    def _():
        drain_out()

    @pl.when(trip >= 1)
    def _():
        drain_out()

    @pl.when(wid == NW - 1)
    def _():
        pltpu.sync_copy(utail_ref, inbuf_u.at[0])
        pltpu.sync_copy(itail_ref, inbuf_i.at[0])
        transpose_block(inbuf_u.at[0], tbuf_u.at[0], TAIL // L)
        transpose_block(inbuf_i.at[0], tbuf_i.at[0], TAIL // L)
        nq = TAIL // 4
        pltpu.sync_copy(tbuf_u.at[0].at[pl.ds(0, nq)],
                        uq_ref.at[pl.ds(NB * 32, nq)])
        pltpu.sync_copy(tbuf_i.at[0].at[pl.ds(0, nq)],
                        iq_ref.at[pl.ds(NB * 32, nq)])


_relayout = functools.partial(
    pl.kernel,
    out_type=(jax.ShapeDtypeStruct((VQ, 2, 2 * D), jnp.float32),
              jax.ShapeDtypeStruct((VQ, 2, 2 * D), jnp.float32)),
    mesh=plsc.VectorSubcoreMesh(core_axis_name="c", subcore_axis_name="s"),
    compiler_params=pltpu.CompilerParams(
        needs_layout_passes=False, use_tc_tiling_on_sc=True),
    scratch_types=[
        pltpu.VMEM((2, D, 128), jnp.float32),
        pltpu.VMEM((2, D, 128), jnp.float32),
        pltpu.VMEM((2, 32, 2, 2 * D), jnp.float32),
        pltpu.VMEM((2, 32, 2, 2 * D), jnp.float32),
        pltpu.SemaphoreType.DMA,
        pltpu.SemaphoreType.DMA,
    ],
)(_relayout_body)


def _gmf_body(users_ref, items_ref, ut_ref, it_ref, wb_ref, out_ref,
              idx_u, idx_i, pr_u, pr_i, bu, bi, out_v, wb_v, sem_u, sem_i):
    c = lax.axis_index("c")
    s = lax.axis_index("s")
    wid = s * NC + c
    base = wid * BPW

    pltpu.sync_copy(users_ref.at[pl.ds(base, BPW)], idx_u)
    pltpu.sync_copy(items_ref.at[pl.ds(base, BPW)], idx_i)
    pltpu.sync_copy(wb_ref, wb_v)

    def quarter(k, carry):
        pr_u[pl.ds(k * L, L)] = idx_u[pl.ds(k * L, L)] >> 2
        pr_i[pl.ds(k * L, L)] = idx_i[pl.ds(k * L, L)] >> 2
        return carry

    lax.fori_loop(0, BPW // L, quarter, 0)

    def fire(j):
        b = j % 2
        cu = pltpu.async_copy(
            ut_ref.at[pr_u.at[pl.ds(j * CHUNK, CHUNK)]], bu.at[b], sem_u)
        ci = pltpu.async_copy(
            it_ref.at[pr_i.at[pl.ds(j * CHUNK, CHUNK)]], bi.at[b], sem_i)
        return cu, ci

    wvecs = [wb_v[pl.ds(k * L, L)] for k in range(D // L)]
    wscal = [wvecs[d // L][d % L] for d in range(D)]
    bias_vec = wb_v[pl.ds(D, L)]
    iota = lax.iota(jnp.int32, L)
    one = jnp.full((L,), 1, jnp.int32)
    sixty4 = jnp.full((L,), D, jnp.int32)

    inflight = fire(0)
    for j in range(NCHUNK):
        nxt = fire(j + 1) if j + 1 < NCHUNK else None
        inflight[0].wait()
        inflight[1].wait()
        b = j % 2
        bu_j = bu.at[b]
        bi_j = bi.at[b]
        for g in range(GPC):
            row16 = iota + (g * L)
            vu = idx_u[pl.ds(j * CHUNK + g * L, L)]
            vi = idx_i[pl.ds(j * CHUNK + g * L, L)]
            su = (vu >> 1) & one
            si = (vi >> 1) & one
            cu0 = (vu & one) * sixty4
            ci0 = (vi & one) * sixty4
            acc = jnp.zeros((L,), jnp.float32)
            for cc in range(D):
                u = plsc.load_gather(bu_j, [row16, su, cu0 + cc])
                v = plsc.load_gather(bi_j, [row16, si, ci0 + cc])
                acc = acc + (u * v) * wscal[cc]
            x = acc + bias_vec
            out_v[pl.ds(j * CHUNK + g * L, L)] = 1.0 / (1.0 + jnp.exp(-x))
        inflight = nxt

    pltpu.sync_copy(out_v, out_ref.at[pl.ds(base, BPW)])


_gmf = functools.partial(
    pl.kernel,
    out_type=jax.ShapeDtypeStruct((B,), jnp.float32),
    mesh=plsc.VectorSubcoreMesh(core_axis_name="c", subcore_axis_name="s"),
    compiler_params=pltpu.CompilerParams(
        needs_layout_passes=False, use_tc_tiling_on_sc=True),
    scratch_types=[
        pltpu.VMEM((BPW,), jnp.int32),
        pltpu.VMEM((BPW,), jnp.int32),
        pltpu.VMEM((BPW,), jnp.int32),
        pltpu.VMEM((BPW,), jnp.int32),
        pltpu.VMEM((2, CHUNK, 2, 2 * D), jnp.float32),
        pltpu.VMEM((2, CHUNK, 2, 2 * D), jnp.float32),
        pltpu.VMEM((BPW,), jnp.float32),
        pltpu.VMEM((D + L,), jnp.float32),
        pltpu.SemaphoreType.DMA,
        pltpu.SemaphoreType.DMA,
    ],
)(_gmf_body)


def kernel(users, items, user_table, item_table, predict_w, predict_b):
    wb = jnp.concatenate(
        [predict_w.reshape(-1), jnp.full((L,), predict_b[0], jnp.float32)])
    utail = jnp.pad(user_table[NB * 128:].T, ((0, 0), (0, 128 - TAIL)))
    itail = jnp.pad(item_table[NB * 128:].T, ((0, 0), (0, 128 - TAIL)))
    ut_q, it_q = _relayout(user_table.T, item_table.T, utail, itail)
    return _gmf(users.astype(jnp.int32), items.astype(jnp.int32),
                ut_q, it_q, wb)


# speedup vs baseline: 3.2328x; 3.2328x over previous
import functools

import jax
import jax.numpy as jnp
from jax import lax
from jax.experimental import pallas as pl
from jax.experimental.pallas import tpu as pltpu
from jax.experimental.pallas import tpu_sc as plsc

B = 16384
D = 64
V = 1000000
VP = V // 2
NC = 2
NS = 16
NW = NC * NS
BPW = B // NW
NCHUNK = 4
CHUNK = BPW // NCHUNK
L = 16
GPC = CHUNK // L

TBLK = 6400
TGRID = (V + TBLK - 1) // TBLK


def _transpose_body(inT_ref, out_ref):
    x = inT_ref[...]
    r = lax.broadcasted_iota(jnp.int32, (D, D), 0)
    c = lax.broadcasted_iota(jnp.int32, (D, D), 1)
    ident = (r == c).astype(jnp.bfloat16)
    xt = lax.dot_general(x.astype(jnp.bfloat16), ident,
                         (((0,), (0,)), ((), ())),
                         preferred_element_type=jnp.float32)
    out_ref[:, 0:D] = xt


_transpose = pl.pallas_call(
    _transpose_body,
    grid=(TGRID,),
    in_specs=[pl.BlockSpec((D, TBLK), lambda k: (0, k))],
    out_specs=pl.BlockSpec((TBLK, 2 * D), lambda k: (k, 0)),
    out_shape=jax.ShapeDtypeStruct((V, 2 * D), jnp.float32),
)


def _gmf_body(users_ref, items_ref, ut_ref, it_ref, wb_ref, out_ref,
              idx_u, idx_i, pr_i, bu, bi, out_v, wb_v, sem_u, sem_i):
    c = lax.axis_index("c")
    s = lax.axis_index("s")
    wid = s * NC + c
    base = wid * BPW

    pltpu.sync_copy(users_ref.at[pl.ds(base, BPW)], idx_u)
    pltpu.sync_copy(items_ref.at[pl.ds(base, BPW)], idx_i)
    pltpu.sync_copy(wb_ref, wb_v)

    def halve(k, carry):
        pr_i[pl.ds(k * L, L)] = idx_i[pl.ds(k * L, L)] >> 1
        return carry

    lax.fori_loop(0, BPW // L, halve, 0)

    def fire(j):
        b = j % 2
        cu = pltpu.async_copy(
            ut_ref.at[idx_u.at[pl.ds(j * CHUNK, CHUNK)]], bu.at[b], sem_u)
        ci = pltpu.async_copy(
            it_ref.at[pr_i.at[pl.ds(j * CHUNK, CHUNK)]], bi.at[b], sem_i)
        return cu, ci

    wvecs = [wb_v[pl.ds(k * L, L)] for k in range(D // L)]
    wscal = [wvecs[d // L][d % L] for d in range(D)]
    bias_vec = wb_v[pl.ds(D, L)]
    iota = lax.iota(jnp.int32, L)
    sixty4 = jnp.full((L,), D, jnp.int32)

    inflight = fire(0)
    for j in range(NCHUNK):
        nxt = fire(j + 1) if j + 1 < NCHUNK else None
        inflight[0].wait()
        inflight[1].wait()
        b = j % 2
        bu_j = bu.at[b]
        bi_j = bi.at[b]
        for g in range(GPC):
            row16 = iota + (g * L)
            pari = (idx_i[pl.ds(j * CHUNK + g * L, L)] & 1) * sixty4
            acc = jnp.zeros((L,), jnp.float32)
            for cc in range(D):
                u = plsc.load_gather(bu_j, [row16, jnp.full((L,), cc, jnp.int32)])
                v = plsc.load_gather(bi_j, [row16, pari + cc])
                acc = acc + (u * v) * wscal[cc]
            x = acc + bias_vec
            out_v[pl.ds(j * CHUNK + g * L, L)] = 1.0 / (1.0 + jnp.exp(-x))
        inflight = nxt

    pltpu.sync_copy(out_v, out_ref.at[pl.ds(base, BPW)])


_gmf = functools.partial(
    pl.kernel,
    out_type=jax.ShapeDtypeStruct((B,), jnp.float32),
    mesh=plsc.VectorSubcoreMesh(core_axis_name="c", subcore_axis_name="s"),
    compiler_params=pltpu.CompilerParams(
        needs_layout_passes=False, use_tc_tiling_on_sc=True),
    scratch_types=[
        pltpu.VMEM((BPW,), jnp.int32),
        pltpu.VMEM((BPW,), jnp.int32),
        pltpu.VMEM((BPW,), jnp.int32),
        pltpu.VMEM((2, CHUNK, 2 * D), jnp.float32),
        pltpu.VMEM((2, CHUNK, 2 * D), jnp.float32),
        pltpu.VMEM((BPW,), jnp.float32),
        pltpu.VMEM((D + L,), jnp.float32),
        pltpu.SemaphoreType.DMA,
        pltpu.SemaphoreType.DMA,
    ],
)(_gmf_body)


def kernel(users, items, user_table, item_table, predict_w, predict_b):
    wb = jnp.concatenate(
        [predict_w.reshape(-1), jnp.full((L,), predict_b[0], jnp.float32)])
    ut_rows = _transpose(user_table.T)
    return _gmf(users.astype(jnp.int32), items.astype(jnp.int32),
                ut_rows, item_table.reshape(VP, 2 * D), wb)
